# baseline (speedup 1.0000x reference)
import jax
import jax.numpy as jnp
from jax import lax
from jax.experimental import pallas as pl
from jax.experimental.pallas import tpu as pltpu
from jax.experimental.pallas import tpu_sc as plsc

_B = 1024
_D = 128
_N = 256
_NTILES = 32
_QPT = _B // _NTILES
_NCHUNK = _N // 16


def _allmin(v):
    lane = lax.broadcasted_iota(jnp.int32, (16,), 0)
    for k in (8, 4, 2, 1):
        v = jnp.minimum(v, v.at[lane ^ k].get(mode="promise_in_bounds"))
    return v


def _sc_bmu(x_hbm, wt_hbm, out_hbm, x_v, wt_v, out_v):
    cid = lax.axis_index("c")
    sid = lax.axis_index("s")
    wid = sid * 2 + cid
    base = wid * _QPT
    pltpu.sync_copy(x_hbm.at[pl.ds(base, _QPT)], x_v)
    pltpu.sync_copy(wt_hbm, wt_v)

    lane = lax.broadcasted_iota(jnp.int32, (16,), 0)
    big_i = jnp.full((16,), 1 << 30, jnp.int32)

    def per_query(qq, bestv, h):
        q = h * 16 + qq
        def gbody(g, tot):
            xg = x_v[q, pl.ds(g * 16, 16)]
            accs = [jnp.zeros((16,), jnp.float32) for _ in range(_NCHUNK)]
            for dd in range(16):
                xv = jnp.full((16,), xg[dd], jnp.float32)
                d = g * 16 + dd
                for cc in range(_NCHUNK):
                    wv = wt_v[d, pl.ds(cc * 16, 16)]
                    diff = wv - xv
                    accs[cc] = accs[cc] + diff * diff
            return tuple(t + a for t, a in zip(tot, accs))
        tot = lax.fori_loop(
            0, 8, gbody,
            tuple(jnp.zeros((16,), jnp.float32) for _ in range(_NCHUNK)))
        tot = list(tot)
        m = tot[0]
        for cc in range(1, _NCHUNK):
            m = jnp.minimum(m, tot[cc])
        mvec = _allmin(m)
        vbest = big_i
        for cc in range(_NCHUNK):
            cand = jnp.where(tot[cc] == mvec, lane + cc * 16, big_i)
            vbest = jnp.minimum(vbest, cand)
        bestall = _allmin(vbest)
        return jnp.where(lane == qq, bestall, bestv)

    half = lane >> 1
    even = (lane & 1) == 0
    for h in range(2):
        bestv = lax.fori_loop(
            0, 16, lambda qq, bv: per_query(qq, bv, h),
            jnp.zeros((16,), jnp.int32))
        rowv = bestv >> 4
        colv = bestv & 15
        for k in range(2):
            idx = half + k * 8
            r = rowv.at[idx].get(mode="promise_in_bounds")
            c = colv.at[idx].get(mode="promise_in_bounds")
            out_v[pl.ds(h * 32 + k * 16, 16)] = jnp.where(even, r, c)

    pltpu.sync_copy(out_v, out_hbm.at[pl.ds(base * 2, _QPT * 2)])


def kernel(x, weights):
    w_t = weights.reshape(_N, _D).T
    sc_call = pl.kernel(
        _sc_bmu,
        out_type=jax.ShapeDtypeStruct((_B * 2,), jnp.int32),
        mesh=plsc.VectorSubcoreMesh(core_axis_name="c", subcore_axis_name="s"),
        scratch_types=[
            pltpu.VMEM((_QPT, _D), jnp.float32),
            pltpu.VMEM((_D, _N), jnp.float32),
            pltpu.VMEM((_QPT * 2,), jnp.int32),
        ],
    )
    return sc_call(x, w_t).reshape(_B, 2)


# speedup vs baseline: 14.3631x; 14.3631x over previous
import jax
import jax.numpy as jnp
from jax.experimental import pallas as pl


def _bmu_kernel(x_ref, w_ref, out_ref):
    x = x_ref[...]
    wt = w_ref[...].T
    wn = jnp.sum(wt * wt, axis=0, keepdims=True)
    xh = x.astype(jnp.bfloat16)
    xr = x - xh.astype(jnp.float32)
    xm = xr.astype(jnp.bfloat16)
    xl = (xr - xm.astype(jnp.float32)).astype(jnp.bfloat16)
    wh = wt.astype(jnp.bfloat16)
    wr = wt - wh.astype(jnp.float32)
    wm = wr.astype(jnp.bfloat16)
    wl = (wr - wm.astype(jnp.float32)).astype(jnp.bfloat16)
    xhm = jnp.concatenate([xh, xm], axis=1)
    xhl = jnp.concatenate([xh, xl], axis=1)
    f32 = jnp.float32
    dots = (jnp.dot(xhm, jnp.concatenate([wh, wm], axis=0),
                    preferred_element_type=f32)
            + jnp.dot(xhm, jnp.concatenate([wm, wh], axis=0),
                      preferred_element_type=f32)
            + jnp.dot(xhl, jnp.concatenate([wl, wh], axis=0),
                      preferred_element_type=f32))
    scores = wn - 2.0 * dots
    m = jnp.min(scores, axis=1, keepdims=True)
    iota = jax.lax.broadcasted_iota(jnp.int32, scores.shape, 1)
    idx = jnp.min(jnp.where(scores == m, iota, scores.shape[1]),
                  axis=1, keepdims=True)
    row = idx // 16
    col = idx - row * 16
    lane = jax.lax.broadcasted_iota(jnp.int32, out_ref.shape, 1)
    out_ref[...] = jnp.where(lane == 0, row, col)


def kernel(x, weights):
    batch, in_size = x.shape
    w_flat = weights.reshape(-1, in_size)
    return pl.pallas_call(
        _bmu_kernel,
        out_shape=jax.ShapeDtypeStruct((batch, 2), jnp.int32),
    )(x, w_flat)


# speedup vs baseline: 14.8829x; 1.0362x over previous
import jax
import jax.numpy as jnp
from jax.experimental import pallas as pl


def _bmu_kernel(x_ref, w_ref, out_ref):
    x = x_ref[...]
    wt = w_ref[...].T
    wn = jnp.sum(wt * wt, axis=0, keepdims=True)
    xh = x.astype(jnp.bfloat16)
    xr = x - xh.astype(jnp.float32)
    xm = xr.astype(jnp.bfloat16)
    xl = (xr - xm.astype(jnp.float32)).astype(jnp.bfloat16)
    wh = wt.astype(jnp.bfloat16)
    wr = wt - wh.astype(jnp.float32)
    wm = wr.astype(jnp.bfloat16)
    wl = (wr - wm.astype(jnp.float32)).astype(jnp.bfloat16)
    xhm = jnp.concatenate([xh, xm], axis=1)
    xhl = jnp.concatenate([xh, xl], axis=1)
    f32 = jnp.float32
    dots = (jnp.dot(xhm, jnp.concatenate([wh, wm], axis=0),
                    preferred_element_type=f32)
            + jnp.dot(xhm, jnp.concatenate([wm, wh], axis=0),
                      preferred_element_type=f32)
            + jnp.dot(xhl, jnp.concatenate([wl, wh], axis=0),
                      preferred_element_type=f32))
    scores = wn - 2.0 * dots
    m = jnp.min(scores, axis=1, keepdims=True)
    iota = jax.lax.broadcasted_iota(jnp.int32, scores.shape, 1)
    idx = jnp.min(jnp.where(scores == m, iota, scores.shape[1]),
                  axis=1, keepdims=True)
    row = jax.lax.shift_right_logical(idx, 4)
    col = idx & 15
    lane = jax.lax.broadcasted_iota(jnp.int32, out_ref.shape, 1)
    out_ref[...] = jnp.where(lane == 0, row, col)


def kernel(x, weights):
    batch, in_size = x.shape
    w_flat = weights.reshape(-1, in_size)
    return pl.pallas_call(
        _bmu_kernel,
        out_shape=jax.ShapeDtypeStruct((batch, 2), jnp.int32),
    )(x, w_flat)
